# Initial kernel scaffold; baseline (speedup 1.0000x reference)
import jax
import jax.numpy as jnp
from jax.experimental import pallas as pl


def kernel(tokenized_text, token_embedding_weight):
    raise NotImplementedError("write your pallas kernel here")


# speedup vs baseline: 1.7188x; 1.7188x over previous
import functools

import jax
import jax.numpy as jnp
from jax import lax
from jax.experimental import pallas as pl
from jax.experimental.pallas import tpu as pltpu
from jax.experimental.pallas import tpu_sc as plsc

LANES = 16
NUM_QUERIES = 16


@functools.lru_cache(maxsize=None)
def _build_sc_kernel(B, CTX, V, D):
    info = plsc.get_sparse_core_info()
    NC, NS = info.num_cores, info.num_subcores
    NW = NC * NS
    assert B % NW == 0
    b_per_w = B // NW
    DV = D // LANES
    inv_ctx = 1.0 / CTX
    ctx_pad = ((CTX + LANES - 1) // LANES) * LANES

    mesh = plsc.VectorSubcoreMesh(core_axis_name="c", subcore_axis_name="s")

    @functools.partial(
        pl.kernel,
        mesh=mesh,
        out_type=jax.ShapeDtypeStruct((B * NUM_QUERIES, D), jnp.float32),
        scratch_types=[
            pltpu.VMEM((b_per_w, ctx_pad), jnp.int32),
            pltpu.VMEM((ctx_pad, D), jnp.float32),
            pltpu.VMEM((NUM_QUERIES, D), jnp.float32),
            pltpu.SemaphoreType.DMA,
        ],
    )
    def k(tok_hbm, table_hbm, out_hbm, idx_v, rows_v, stage_v, sem):
        wid = lax.axis_index("s") * NC + lax.axis_index("c")
        base_ex = wid * b_per_w
        pltpu.sync_copy(tok_hbm.at[pl.ds(base_ex, b_per_w)], idx_v)

        def ebody(e, carry):
            pltpu.async_copy(table_hbm.at[idx_v.at[e]], rows_v, sem).wait()

            def rbody(r, acc):
                return tuple(
                    acc[j] + rows_v[r, pl.ds(j * LANES, LANES)]
                    for j in range(DV)
                )

            acc0 = tuple(rows_v[0, pl.ds(j * LANES, LANES)] for j in range(DV))
            acc = lax.fori_loop(1, CTX, rbody, acc0)
            mean = [acc[j] * inv_ctx for j in range(DV)]

            def qbody(q, c):
                for j in range(DV):
                    stage_v[q, pl.ds(j * LANES, LANES)] = mean[j]
                return c

            lax.fori_loop(0, NUM_QUERIES, qbody, 0)
            pltpu.sync_copy(
                stage_v, out_hbm.at[pl.ds((base_ex + e) * NUM_QUERIES, NUM_QUERIES)]
            )
            return carry

        lax.fori_loop(0, b_per_w, ebody, 0)

    return k


def kernel(tokenized_text, token_embedding_weight):
    B, CTX = tokenized_text.shape
    V, D = token_embedding_weight.shape
    tok = tokenized_text.astype(jnp.int32)
    ctx_pad = ((CTX + LANES - 1) // LANES) * LANES
    if ctx_pad != CTX:
        tok = jnp.pad(tok, ((0, 0), (0, ctx_pad - CTX)))
    k = _build_sc_kernel(B, CTX, V, D)
    out = k(tok, token_embedding_weight)
    return out.reshape(B, NUM_QUERIES, D)
